# baseline (speedup 1.0000x reference)
import jax
import jax.numpy as jnp
from jax.experimental import pallas as pl
from jax.experimental.pallas import tpu as pltpu

NUM_EMBEDDINGS = 1024
EMBED_DIM = 64
BLK = 2048
JT = 128
NT = NUM_EMBEDDINGS // JT


def _l2n(v):
    return v * jax.lax.rsqrt((v * v).sum(axis=-1, keepdims=True) + 1e-12)


def _prep_body(cb_ref, proj_ref, cbp_ref, cb2_ref, cbn_ref):
    cbp = jax.lax.dot_general(
        cb_ref[...], proj_ref[...], (((1,), (0,)), ((), ())),
        preferred_element_type=jnp.float32)
    cbp = _l2n(cbp)
    cbp_ref[...] = cbp
    cb2_ref[...] = (cbp * cbp).sum(axis=1, keepdims=True).reshape(1, -1)
    cbn_ref[...] = _l2n(cb_ref[...])


def _vq_body(x_ref, proj_ref, cbp_ref, cb2_ref, cbn_ref, disc_ref, quant_ref):
    xp = jax.lax.dot_general(
        x_ref[...], proj_ref[...], (((1,), (0,)), ((), ())),
        preferred_element_type=jnp.float32)
    xp = _l2n(xp)
    x2 = (xp * xp).sum(axis=1, keepdims=True)

    run_min = None
    run_j = None
    lane = jax.lax.broadcasted_iota(jnp.int32, (BLK, JT), 1)
    for t in range(NT):
        dots_t = jax.lax.dot_general(
            xp, cbp_ref[t * JT:(t + 1) * JT, :], (((1,), (1,)), ((), ())),
            preferred_element_type=jnp.float32)
        d_t = (x2 + (-2.0) * dots_t) + cb2_ref[:, t * JT:(t + 1) * JT]
        if t == 0:
            run_min = d_t
            run_j = lane
        else:
            pred = d_t < run_min
            run_min = jnp.where(pred, d_t, run_min)
            run_j = jnp.where(pred, lane + t * JT, run_j)

    m = jnp.min(run_min, axis=1, keepdims=True)
    idx = jnp.min(jnp.where(run_min == m, run_j, NUM_EMBEDDINGS),
                  axis=1, keepdims=True)

    q = jnp.zeros((BLK, EMBED_DIM), jnp.float32)
    for t in range(NT):
        disc_t = (lane + t * JT == idx).astype(jnp.float32)
        disc_ref[:, t * JT:(t + 1) * JT] = disc_t
        q = q + jax.lax.dot_general(
            disc_t, cbn_ref[t * JT:(t + 1) * JT, :], (((1,), (0,)), ((), ())),
            preferred_element_type=jnp.float32)
    quant_ref[...] = q


def kernel(x, codebook, proj_kernel):
    x_flat = x.reshape(-1, EMBED_DIM)
    n = x_flat.shape[0]

    cbp, cb2, cbn = pl.pallas_call(
        _prep_body,
        out_shape=[
            jax.ShapeDtypeStruct((NUM_EMBEDDINGS, EMBED_DIM), jnp.float32),
            jax.ShapeDtypeStruct((1, NUM_EMBEDDINGS), jnp.float32),
            jax.ShapeDtypeStruct((NUM_EMBEDDINGS, EMBED_DIM), jnp.float32),
        ],
    )(codebook, proj_kernel)

    disc, quant = pl.pallas_call(
        _vq_body,
        grid=(n // BLK,),
        in_specs=[
            pl.BlockSpec((BLK, EMBED_DIM), lambda i: (i, 0)),
            pl.BlockSpec((EMBED_DIM, EMBED_DIM), lambda i: (0, 0)),
            pl.BlockSpec((NUM_EMBEDDINGS, EMBED_DIM), lambda i: (0, 0)),
            pl.BlockSpec((1, NUM_EMBEDDINGS), lambda i: (0, 0)),
            pl.BlockSpec((NUM_EMBEDDINGS, EMBED_DIM), lambda i: (0, 0)),
        ],
        out_specs=[
            pl.BlockSpec((BLK, NUM_EMBEDDINGS), lambda i: (i, 0)),
            pl.BlockSpec((BLK, EMBED_DIM), lambda i: (i, 0)),
        ],
        out_shape=[
            jax.ShapeDtypeStruct((n, NUM_EMBEDDINGS), jnp.float32),
            jax.ShapeDtypeStruct((n, EMBED_DIM), jnp.float32),
        ],
    )(x_flat, proj_kernel, cbp, cb2, cbn)
    return disc, quant.reshape(x.shape[:-1] + (EMBED_DIM,))
